# baseline (speedup 1.0000x reference)
import functools

import jax
import jax.numpy as jnp
from jax import lax
from jax.experimental import pallas as pl
from jax.experimental.pallas import tpu as pltpu
from jax.experimental.pallas import tpu_sc as plsc

N = 10000
E = 160000
D = 256
DH = 128
NC = 2
NS = 16
NW = NC * NS
CH = 128
NCHG = E // CH
NCHT = (NCHG // NS) & ~1
NEXTRA = NCHG - NCHT * NS
RPT = 640
NP = NS * RPT


def _sc_aggregate(x2, packed, z2d, z1d):
    mesh = plsc.VectorSubcoreMesh(core_axis_name="c", subcore_axis_name="s")

    @functools.partial(
        pl.kernel,
        out_type=(
            jax.ShapeDtypeStruct((NC, NP, DH), jnp.float32),
            jax.ShapeDtypeStruct((NW * NP,), jnp.float32),
        ),
        mesh=mesh,
        compiler_params=pltpu.CompilerParams(needs_layout_passes=False),
        scratch_types=[
            pltpu.VMEM((2, 2, CH), jnp.int32),
            pltpu.VMEM((2, CH, DH), jnp.float32),
            pltpu.VMEM((NP,), jnp.float32),
            pltpu.VMEM_SHARED((NP, DH), jnp.float32),
            pltpu.SemaphoreType.DMA,
            pltpu.SemaphoreType.DMA,
        ],
    )
    def k(x2_hbm, packed_hbm, z2d_hbm, z1d_hbm, out_hbm, deg_hbm,
          idx2, rows, degl, acc, gsem0, gsem1):
        cid = lax.axis_index("c")
        sid = lax.axis_index("s")
        wid = cid * NS + sid

        r0 = sid * RPT
        pltpu.sync_copy(z2d_hbm, acc.at[pl.ds(r0, RPT)])
        pltpu.sync_copy(z1d_hbm, degl)
        plsc.subcore_barrier()

        ones = jnp.full((16,), 1.0, dtype=jnp.float32)
        gsems = (gsem0, gsem1)

        def start_gather(j, b):
            c = jnp.minimum(j, NCHT - 1) * NS + sid
            pltpu.sync_copy(packed_hbm.at[cid, c], idx2.at[b])
            pltpu.async_copy(x2_hbm.at[idx2.at[b, 0]], rows.at[b], gsems[b])

        def wait_gather(b):
            pltpu.make_async_copy(x2_hbm.at[idx2.at[b, 0]], rows.at[b],
                                  gsems[b]).wait()

        def consume(b):
            for i in range(CH // 16):
                plsc.addupdate_scatter(degl, [idx2[b, 1, pl.ds(i * 16, 16)]],
                                       ones)
            pltpu.sync_copy(rows.at[b], acc.at[idx2.at[b, 1]], add=True)

        start_gather(0, 0)

        def pair(jj, carry):
            j0 = 2 * jj
            start_gather(j0 + 1, 1)
            wait_gather(0)
            consume(0)
            start_gather(j0 + 2, 0)
            wait_gather(1)
            consume(1)
            return carry

        lax.fori_loop(0, NCHT // 2, pair, 0, unroll=False)
        wait_gather(0)

        @pl.when(sid < NEXTRA)
        def _():
            c = NCHT * NS + sid
            pltpu.sync_copy(packed_hbm.at[cid, c], idx2.at[0])
            pltpu.async_copy(x2_hbm.at[idx2.at[0, 0]], rows.at[0],
                             gsem0).wait()
            consume(0)

        plsc.subcore_barrier()
        pltpu.sync_copy(acc.at[pl.ds(r0, RPT)], out_hbm.at[cid, pl.ds(r0, RPT)])
        pltpu.sync_copy(degl, deg_hbm.at[pl.ds(wid * NP, NP)])

    return k(x2, packed, z2d, z1d)


def _tc_self(x, wst):
    BR = 1280

    def body(x_ref, wst_ref, o_ref):
        o_ref[...] = jnp.dot(x_ref[...], wst_ref[...],
                             preferred_element_type=jnp.float32)

    return pl.pallas_call(
        body,
        grid=(pl.cdiv(N, BR),),
        in_specs=[
            pl.BlockSpec((BR, D), lambda i: (i, 0)),
            pl.BlockSpec((D, D), lambda i: (0, 0)),
        ],
        out_specs=pl.BlockSpec((BR, D), lambda i: (i, 0)),
        out_shape=jax.ShapeDtypeStruct((N, D), jnp.float32),
    )(x, wst)


def _tc_finish(hs, wnt, summed, degp):
    BR = 1280

    def body(hs_ref, wnt_ref, s_ref, degp_ref, o_ref):
        deg = jnp.sum(degp_ref[...], axis=1, keepdims=True) * 0.5
        scale = jnp.maximum(deg, 1.0)
        keep = deg > 0.0
        m0 = jnp.where(keep, s_ref[0] / scale, 0.0)
        m1 = jnp.where(keep, s_ref[1] / scale, 0.0)
        o_ref[...] = (
            hs_ref[...]
            + jnp.dot(m0, wnt_ref[:DH, :], preferred_element_type=jnp.float32)
            + jnp.dot(m1, wnt_ref[DH:, :], preferred_element_type=jnp.float32)
        )

    return pl.pallas_call(
        body,
        grid=(pl.cdiv(N, BR),),
        in_specs=[
            pl.BlockSpec((BR, D), lambda i: (i, 0)),
            pl.BlockSpec((D, D), lambda i: (0, 0)),
            pl.BlockSpec((NC, BR, DH), lambda i: (0, i, 0)),
            pl.BlockSpec((BR, NW), lambda i: (i, 0)),
        ],
        out_specs=pl.BlockSpec((BR, D), lambda i: (i, 0)),
        out_shape=jax.ShapeDtypeStruct((N, D), jnp.float32),
    )(hs, wnt, summed, degp)


def kernel(x, edge_index, W_self, W_neigh):
    ei = edge_index.astype(jnp.int32)
    src2 = ei[0] * 2
    dst = ei[1]
    packed = jnp.stack(
        [jnp.stack([src2, dst]), jnp.stack([src2 + 1, dst])]
    ).reshape(NC, 2, NCHG, CH).transpose(0, 2, 1, 3)
    x2 = x.reshape(NC * N, DH)
    z2d = jnp.zeros((RPT, DH), jnp.float32)
    z1d = jnp.zeros((NP,), jnp.float32)
    summed, degp = _sc_aggregate(x2, packed, z2d, z1d)
    hs = _tc_self(x, W_self.T)
    degp2 = degp.reshape(NW, NP).T
    return _tc_finish(hs, W_neigh.T, summed, degp2)


# speedup vs baseline: 1.0200x; 1.0200x over previous
import functools

import jax
import jax.numpy as jnp
from jax import lax
from jax.experimental import pallas as pl
from jax.experimental.pallas import tpu as pltpu
from jax.experimental.pallas import tpu_sc as plsc

N = 10000
E = 160000
D = 256
DH = 128
NC = 2
NS = 16
NW = NC * NS
CH = 128
NCHG = E // CH
NCHT = (NCHG // NS) & ~1
NEXTRA = NCHG - NCHT * NS
RPT = 640
NP = NS * RPT


def _sc_aggregate(x2, packed, z2d, z1d):
    mesh = plsc.VectorSubcoreMesh(core_axis_name="c", subcore_axis_name="s")

    @functools.partial(
        pl.kernel,
        out_type=(
            jax.ShapeDtypeStruct((NC, NP, DH), jnp.float32),
            jax.ShapeDtypeStruct((NW * NP,), jnp.float32),
        ),
        mesh=mesh,
        compiler_params=pltpu.CompilerParams(needs_layout_passes=False),
        scratch_types=[
            pltpu.VMEM((2, 2, CH), jnp.int32),
            pltpu.VMEM((2, CH, DH), jnp.float32),
            pltpu.VMEM((NP,), jnp.float32),
            pltpu.VMEM_SHARED((NP, DH), jnp.float32),
            pltpu.SemaphoreType.DMA,
            pltpu.SemaphoreType.DMA,
        ],
    )
    def k(x2_hbm, packed_hbm, z2d_hbm, z1d_hbm, out_hbm, deg_hbm,
          idx2, rows, degl, acc, gsem0, gsem1):
        cid = lax.axis_index("c")
        sid = lax.axis_index("s")
        wid = cid * NS + sid

        r0 = sid * RPT
        pltpu.sync_copy(z2d_hbm, acc.at[pl.ds(r0, RPT)])
        pltpu.sync_copy(z1d_hbm, degl)
        plsc.subcore_barrier()

        ones = jnp.full((16,), 1.0, dtype=jnp.float32)
        gsems = (gsem0, gsem1)

        def start_gather(j, b):
            c = jnp.minimum(j, NCHT - 1) * NS + sid
            pltpu.sync_copy(packed_hbm.at[cid, c], idx2.at[b])
            pltpu.async_copy(x2_hbm.at[idx2.at[b, 0]], rows.at[b], gsems[b])

        def wait_gather(b):
            pltpu.make_async_copy(x2_hbm.at[idx2.at[b, 0]], rows.at[b],
                                  gsems[b]).wait()

        def update_deg(b):
            for i in range(CH // 16):
                plsc.addupdate_scatter(degl, [idx2[b, 1, pl.ds(i * 16, 16)]],
                                       ones)

        def consume(b):
            pltpu.sync_copy(rows.at[b], acc.at[idx2.at[b, 1]], add=True)

        start_gather(0, 0)

        def pair(jj, carry):
            j0 = 2 * jj
            start_gather(j0 + 1, 1)
            update_deg(0)
            wait_gather(0)
            consume(0)
            start_gather(j0 + 2, 0)
            update_deg(1)
            wait_gather(1)
            consume(1)
            return carry

        lax.fori_loop(0, NCHT // 2, pair, 0, unroll=False)
        wait_gather(0)

        @pl.when(sid < NEXTRA)
        def _():
            c = NCHT * NS + sid
            pltpu.sync_copy(packed_hbm.at[cid, c], idx2.at[0])
            pltpu.async_copy(x2_hbm.at[idx2.at[0, 0]], rows.at[0],
                             gsem0).wait()
            update_deg(0)
            consume(0)

        plsc.subcore_barrier()
        pltpu.sync_copy(acc.at[pl.ds(r0, RPT)], out_hbm.at[cid, pl.ds(r0, RPT)])
        pltpu.sync_copy(degl, deg_hbm.at[pl.ds(wid * NP, NP)])

    return k(x2, packed, z2d, z1d)


def _tc_finish(x, wst, wnt, summed, degp):
    BR = 1280

    def body(x_ref, wst_ref, wnt_ref, s_ref, degp_ref, o_ref):
        deg = jnp.sum(degp_ref[...], axis=1, keepdims=True) * 0.5
        scale = jnp.maximum(deg, 1.0)
        keep = deg > 0.0
        m0 = jnp.where(keep, s_ref[0] / scale, 0.0)
        m1 = jnp.where(keep, s_ref[1] / scale, 0.0)
        o_ref[...] = (
            jnp.dot(x_ref[...], wst_ref[...], preferred_element_type=jnp.float32)
            + jnp.dot(m0, wnt_ref[:DH, :], preferred_element_type=jnp.float32)
            + jnp.dot(m1, wnt_ref[DH:, :], preferred_element_type=jnp.float32)
        )

    return pl.pallas_call(
        body,
        grid=(pl.cdiv(N, BR),),
        in_specs=[
            pl.BlockSpec((BR, D), lambda i: (i, 0)),
            pl.BlockSpec((D, D), lambda i: (0, 0)),
            pl.BlockSpec((D, D), lambda i: (0, 0)),
            pl.BlockSpec((NC, BR, DH), lambda i: (0, i, 0)),
            pl.BlockSpec((BR, NW), lambda i: (i, 0)),
        ],
        out_specs=pl.BlockSpec((BR, D), lambda i: (i, 0)),
        out_shape=jax.ShapeDtypeStruct((N, D), jnp.float32),
    )(x, wst, wnt, summed, degp)


def kernel(x, edge_index, W_self, W_neigh):
    ei = edge_index.astype(jnp.int32)
    src2 = ei[0] * 2
    dst = ei[1]
    packed = jnp.stack(
        [jnp.stack([src2, dst]), jnp.stack([src2 + 1, dst])]
    ).reshape(NC, 2, NCHG, CH).transpose(0, 2, 1, 3)
    x2 = x.reshape(NC * N, DH)
    z2d = jnp.zeros((RPT, DH), jnp.float32)
    z1d = jnp.zeros((NP,), jnp.float32)
    summed, degp = _sc_aggregate(x2, packed, z2d, z1d)
    degp2 = degp.reshape(NW, NP).T
    return _tc_finish(x, W_self.T, W_neigh.T, summed, degp2)
